# baseline (speedup 1.0000x reference)
import jax
import jax.numpy as jnp
from jax import lax
from jax.experimental import pallas as pl
from jax.experimental.pallas import tpu as pltpu
from jax.experimental.pallas import tpu_sc as plsc

BATCH = 4096
D = 4096
SIZE = 64
NC = 2
NS = 16
NW = NC * NS
RPW = BATCH // NW
CH = 8
NCHUNK = RPW // CH
NPAIR = NCHUNK // 2
L = 16
UNROLL = 8
NGROUP = D // (UNROLL * L)
_BIG = 1 << 30


def _row_kernel(s_hbm, p0_hbm, p1_hbm, buf0, buf1, o0, o1, sem0, sem1,
                osem0, osem1):
  wid = lax.axis_index("s") * NC + lax.axis_index("c")
  base = wid * RPW
  iota = lax.iota(jnp.int32, L)

  def chunk_copy(g, buf, sem):
    return pltpu.make_async_copy(
        s_hbm.at[pl.ds(base + g * CH, CH)], buf, sem)

  def process_chunk(buf, rowbase_in_pair, rr, accs):
    acc0, acc1 = accs

    def step(i, carry):
      best, besti = carry
      ibase = jnp.broadcast_to(i, (L,))
      for u in range(UNROLL):
        v = buf[rr, pl.ds((i * UNROLL + u) * L, L)]
        m = v > best
        best = jnp.where(m, v, best)
        besti = jnp.where(m, ibase, besti)
      return best, besti

    best, besti = lax.fori_loop(
        0, NGROUP, step,
        (jnp.full((L,), -jnp.inf, jnp.float32), jnp.zeros((L,), jnp.int32)))

    m = best
    for k in (8, 4, 2, 1):
      m = jnp.maximum(m, m.at[iota ^ k].get(mode="promise_in_bounds"))
    candg = jnp.where(best == m, besti, _BIG)
    for k in (8, 4, 2, 1):
      candg = jnp.minimum(candg,
                          candg.at[iota ^ k].get(mode="promise_in_bounds"))
    gstar = candg[0]
    gcol = gstar * (UNROLL * L)
    pos = jnp.full((L,), _BIG, jnp.int32)
    for u in range(UNROLL):
      v = buf[rr, pl.ds(gcol + u * L, L)]
      pos = jnp.minimum(pos, jnp.where(v == m, iota + u * L, pos))
    for k in (8, 4, 2, 1):
      pos = jnp.minimum(pos, pos.at[iota ^ k].get(mode="promise_in_bounds"))
    idx = pos + jnp.broadcast_to(gcol, (L,))

    at_top = idx < SIZE
    at_left = (idx > 0) & ((idx & (SIZE - 1)) == 0)
    p0 = jnp.where(at_top, 0.0, jnp.where(at_left, 1.0, 0.5))
    p1 = jnp.where(at_top, 1.0, jnp.where(at_left, 0.0, 0.5))

    lanemask = iota == rowbase_in_pair + rr
    acc0 = jnp.where(lanemask, p0, acc0)
    acc1 = jnp.where(lanemask, p1, acc1)
    return acc0, acc1

  chunk_copy(0, buf0, sem0).start()
  chunk_copy(1, buf1, sem1).start()

  def pair(gp, carry):
    chunk_copy(2 * gp, buf0, sem0).wait()
    accs = (jnp.zeros((L,), jnp.float32), jnp.zeros((L,), jnp.float32))
    accs = lax.fori_loop(
        0, CH, lambda rr, a: process_chunk(buf0, 0, rr, a), accs)

    @pl.when(gp + 1 < NPAIR)
    def _():
      chunk_copy(2 * gp + 2, buf0, sem0).start()

    chunk_copy(2 * gp + 1, buf1, sem1).wait()
    accs = lax.fori_loop(
        0, CH, lambda rr, a: process_chunk(buf1, CH, rr, a), accs)

    @pl.when(gp + 1 < NPAIR)
    def _():
      chunk_copy(2 * gp + 3, buf1, sem1).start()

    o0[pl.ds(gp * 2 * CH, L)] = accs[0]
    o1[pl.ds(gp * 2 * CH, L)] = accs[1]
    return carry

  lax.fori_loop(0, NPAIR, pair, jnp.int32(0))

  pltpu.make_async_copy(o0, p0_hbm.at[pl.ds(base, RPW)], osem0).start()
  pltpu.make_async_copy(o1, p1_hbm.at[pl.ds(base, RPW)], osem1).start()
  pltpu.make_async_copy(o0, p0_hbm.at[pl.ds(base, RPW)], osem0).wait()
  pltpu.make_async_copy(o1, p1_hbm.at[pl.ds(base, RPW)], osem1).wait()


@jax.jit
def kernel(s):
  p0, p1 = pl.kernel(
      _row_kernel,
      out_type=(jax.ShapeDtypeStruct((BATCH,), jnp.float32),
                jax.ShapeDtypeStruct((BATCH,), jnp.float32)),
      mesh=plsc.VectorSubcoreMesh(core_axis_name="c", subcore_axis_name="s"),
      scratch_types=[
          pltpu.VMEM((CH, D), jnp.float32),
          pltpu.VMEM((CH, D), jnp.float32),
          pltpu.VMEM((RPW,), jnp.float32),
          pltpu.VMEM((RPW,), jnp.float32),
          pltpu.SemaphoreType.DMA,
          pltpu.SemaphoreType.DMA,
          pltpu.SemaphoreType.DMA,
          pltpu.SemaphoreType.DMA,
      ],
  )(s)
  zero = jnp.zeros((BATCH, 1), jnp.float32)
  return jnp.concatenate([p0[:, None], p1[:, None], zero], axis=1)


# speedup vs baseline: 1.2741x; 1.2741x over previous
import jax
import jax.numpy as jnp
from jax import lax
from jax.experimental import pallas as pl
from jax.experimental.pallas import tpu as pltpu
from jax.experimental.pallas import tpu_sc as plsc

BATCH = 4096
D = 4096
SIZE = 64
NC = 2
NS = 16
NW = NC * NS
RPW = BATCH // NW
CH = 8
NCHUNK = RPW // CH
NPAIR = NCHUNK // 2
L = 16
UNROLL = 16
NGROUP = D // (UNROLL * L)
_BIG = 1 << 30


def _row_kernel(s_hbm, p0_hbm, p1_hbm, buf0, buf1, o0, o1, sem0, sem1,
                osem0, osem1):
  wid = lax.axis_index("s") * NC + lax.axis_index("c")
  base = wid * RPW
  iota = lax.iota(jnp.int32, L)

  def chunk_copy(g, buf, sem):
    return pltpu.make_async_copy(
        s_hbm.at[pl.ds(base + g * CH, CH)], buf, sem)

  def process_chunk(buf, rowbase_in_pair, rr, accs):
    acc0, acc1 = accs

    def step(i, carry):
      ba, ia, bb, ib = carry
      ibase = jnp.broadcast_to(i, (L,))
      for u in range(0, UNROLL, 2):
        va = buf[rr, pl.ds((i * UNROLL + u) * L, L)]
        vb = buf[rr, pl.ds((i * UNROLL + u + 1) * L, L)]
        ma = va > ba
        mb = vb > bb
        ba = jnp.where(ma, va, ba)
        ia = jnp.where(ma, ibase, ia)
        bb = jnp.where(mb, vb, bb)
        ib = jnp.where(mb, ibase, ib)
      return ba, ia, bb, ib

    ninf = jnp.full((L,), -jnp.inf, jnp.float32)
    zi = jnp.zeros((L,), jnp.int32)
    ba, ia, bb, ib = lax.fori_loop(0, NGROUP, step, (ninf, zi, ninf, zi))
    takeb = (bb > ba) | ((bb == ba) & (ib < ia))
    best = jnp.where(takeb, bb, ba)
    besti = jnp.where(takeb, ib, ia)

    m = best
    for k in (8, 4, 2, 1):
      m = jnp.maximum(m, m.at[iota ^ k].get(mode="promise_in_bounds"))
    candg = jnp.where(best == m, besti, _BIG)
    for k in (8, 4, 2, 1):
      candg = jnp.minimum(candg,
                          candg.at[iota ^ k].get(mode="promise_in_bounds"))
    gstar = candg[0]
    gcol = gstar * (UNROLL * L)
    pos = jnp.full((L,), _BIG, jnp.int32)
    for u in range(UNROLL):
      v = buf[rr, pl.ds(gcol + u * L, L)]
      pos = jnp.minimum(pos, jnp.where(v == m, iota + u * L, pos))
    for k in (8, 4, 2, 1):
      pos = jnp.minimum(pos, pos.at[iota ^ k].get(mode="promise_in_bounds"))
    idx = pos + jnp.broadcast_to(gcol, (L,))

    at_top = idx < SIZE
    at_left = (idx > 0) & ((idx & (SIZE - 1)) == 0)
    p0 = jnp.where(at_top, 0.0, jnp.where(at_left, 1.0, 0.5))
    p1 = jnp.where(at_top, 1.0, jnp.where(at_left, 0.0, 0.5))

    lanemask = iota == rowbase_in_pair + rr
    acc0 = jnp.where(lanemask, p0, acc0)
    acc1 = jnp.where(lanemask, p1, acc1)
    return acc0, acc1

  chunk_copy(0, buf0, sem0).start()
  chunk_copy(1, buf1, sem1).start()

  def pair(gp, carry):
    chunk_copy(2 * gp, buf0, sem0).wait()
    accs = (jnp.zeros((L,), jnp.float32), jnp.zeros((L,), jnp.float32))
    accs = lax.fori_loop(
        0, CH, lambda rr, a: process_chunk(buf0, 0, rr, a), accs)

    @pl.when(gp + 1 < NPAIR)
    def _():
      chunk_copy(2 * gp + 2, buf0, sem0).start()

    chunk_copy(2 * gp + 1, buf1, sem1).wait()
    accs = lax.fori_loop(
        0, CH, lambda rr, a: process_chunk(buf1, CH, rr, a), accs)

    @pl.when(gp + 1 < NPAIR)
    def _():
      chunk_copy(2 * gp + 3, buf1, sem1).start()

    o0[pl.ds(gp * 2 * CH, L)] = accs[0]
    o1[pl.ds(gp * 2 * CH, L)] = accs[1]
    return carry

  lax.fori_loop(0, NPAIR, pair, jnp.int32(0))

  pltpu.make_async_copy(o0, p0_hbm.at[pl.ds(base, RPW)], osem0).start()
  pltpu.make_async_copy(o1, p1_hbm.at[pl.ds(base, RPW)], osem1).start()
  pltpu.make_async_copy(o0, p0_hbm.at[pl.ds(base, RPW)], osem0).wait()
  pltpu.make_async_copy(o1, p1_hbm.at[pl.ds(base, RPW)], osem1).wait()


@jax.jit
def kernel(s):
  p0, p1 = pl.kernel(
      _row_kernel,
      out_type=(jax.ShapeDtypeStruct((BATCH,), jnp.float32),
                jax.ShapeDtypeStruct((BATCH,), jnp.float32)),
      mesh=plsc.VectorSubcoreMesh(core_axis_name="c", subcore_axis_name="s"),
      scratch_types=[
          pltpu.VMEM((CH, D), jnp.float32),
          pltpu.VMEM((CH, D), jnp.float32),
          pltpu.VMEM((RPW,), jnp.float32),
          pltpu.VMEM((RPW,), jnp.float32),
          pltpu.SemaphoreType.DMA,
          pltpu.SemaphoreType.DMA,
          pltpu.SemaphoreType.DMA,
          pltpu.SemaphoreType.DMA,
      ],
  )(s)
  zero = jnp.zeros((BATCH, 1), jnp.float32)
  return jnp.concatenate([p0[:, None], p1[:, None], zero], axis=1)


# speedup vs baseline: 1.5831x; 1.2425x over previous
import functools

import jax
import jax.numpy as jnp
from jax import lax
from jax.experimental import pallas as pl
from jax.experimental.pallas import tpu as pltpu
from jax.experimental.pallas import tpu_sc as plsc

BATCH = 4096
D = 4096
SIZE = 64
NC = 2
NS = 16
NW = NC * NS
BSC = 1536
RPW = BSC // NW
CH = 8
NCHUNK = RPW // CH
NPAIR = NCHUNK // 2
L = 16
UNROLL = 16
NGROUP = D // (UNROLL * L)
TCB = 256
_BIG = 1 << 30


def _row_kernel(s_hbm, out_hbm, buf0, buf1, o3, sem0, sem1, osem):
  wid = lax.axis_index("s") * NC + lax.axis_index("c")
  base = wid * RPW
  iota = lax.iota(jnp.int32, L)

  def chunk_copy(g, buf, sem):
    return pltpu.make_async_copy(
        s_hbm.at[pl.ds(base + g * CH, CH)], buf, sem)

  def process_chunk(buf, rowbase_in_pair, rr, accs):
    acc0, acc1 = accs

    def step(i, carry):
      ba, ia, bb, ib = carry
      ibase = jnp.broadcast_to(i, (L,))
      for u in range(0, UNROLL, 2):
        va = buf[rr, pl.ds((i * UNROLL + u) * L, L)]
        vb = buf[rr, pl.ds((i * UNROLL + u + 1) * L, L)]
        ma = va > ba
        mb = vb > bb
        ba = jnp.where(ma, va, ba)
        ia = jnp.where(ma, ibase, ia)
        bb = jnp.where(mb, vb, bb)
        ib = jnp.where(mb, ibase, ib)
      return ba, ia, bb, ib

    ninf = jnp.full((L,), -jnp.inf, jnp.float32)
    zi = jnp.zeros((L,), jnp.int32)
    ba, ia, bb, ib = lax.fori_loop(0, NGROUP, step, (ninf, zi, ninf, zi))
    takeb = (bb > ba) | ((bb == ba) & (ib < ia))
    best = jnp.where(takeb, bb, ba)
    besti = jnp.where(takeb, ib, ia)

    m = best
    for k in (8, 4, 2, 1):
      m = jnp.maximum(m, m.at[iota ^ k].get(mode="promise_in_bounds"))
    candg = jnp.where(best == m, besti, _BIG)
    for k in (8, 4, 2, 1):
      candg = jnp.minimum(candg,
                          candg.at[iota ^ k].get(mode="promise_in_bounds"))
    gstar = candg[0]
    gcol = gstar * (UNROLL * L)
    pos_a = jnp.full((L,), _BIG, jnp.int32)
    pos_b = jnp.full((L,), _BIG, jnp.int32)
    for u in range(0, UNROLL, 2):
      va = buf[rr, pl.ds(gcol + u * L, L)]
      vb = buf[rr, pl.ds(gcol + (u + 1) * L, L)]
      pos_a = jnp.minimum(pos_a, jnp.where(va == m, iota + u * L, pos_a))
      pos_b = jnp.minimum(pos_b,
                          jnp.where(vb == m, iota + (u + 1) * L, pos_b))
    pos = jnp.minimum(pos_a, pos_b)
    for k in (8, 4, 2, 1):
      pos = jnp.minimum(pos, pos.at[iota ^ k].get(mode="promise_in_bounds"))
    idx = pos + jnp.broadcast_to(gcol, (L,))

    at_top = idx < SIZE
    at_left = (idx > 0) & ((idx & (SIZE - 1)) == 0)
    p0 = jnp.where(at_top, 0.0, jnp.where(at_left, 1.0, 0.5))
    p1 = jnp.where(at_top, 1.0, jnp.where(at_left, 0.0, 0.5))

    lanemask = iota == rowbase_in_pair + rr
    acc0 = jnp.where(lanemask, p0, acc0)
    acc1 = jnp.where(lanemask, p1, acc1)
    return acc0, acc1

  chunk_copy(0, buf0, sem0).start()
  chunk_copy(1, buf1, sem1).start()

  def pair(gp, carry):
    chunk_copy(2 * gp, buf0, sem0).wait()
    accs = (jnp.zeros((L,), jnp.float32), jnp.zeros((L,), jnp.float32))
    accs = lax.fori_loop(
        0, CH, lambda rr, a: process_chunk(buf0, 0, rr, a), accs)

    @pl.when(gp + 1 < NPAIR)
    def _():
      chunk_copy(2 * gp + 2, buf0, sem0).start()

    chunk_copy(2 * gp + 1, buf1, sem1).wait()
    accs = lax.fori_loop(
        0, CH, lambda rr, a: process_chunk(buf1, CH, rr, a), accs)

    @pl.when(gp + 1 < NPAIR)
    def _():
      chunk_copy(2 * gp + 3, buf1, sem1).start()

    acc0, acc1 = accs
    for j in range(3):
      q = iota + (L * j)
      rowsel = (q * 21846) >> 16
      colsel = q - rowsel * 3
      g0 = acc0.at[rowsel].get(mode="promise_in_bounds")
      g1 = acc1.at[rowsel].get(mode="promise_in_bounds")
      outj = jnp.where(colsel == 0, g0, jnp.where(colsel == 1, g1, 0.0))
      o3[pl.ds(gp * (2 * CH * 3) + j * L, L)] = outj
    return carry

  lax.fori_loop(0, NPAIR, pair, jnp.int32(0))

  pltpu.make_async_copy(
      o3, out_hbm.at[pl.ds(base * 3, RPW * 3)], osem).start()
  pltpu.make_async_copy(
      o3, out_hbm.at[pl.ds(base * 3, RPW * 3)], osem).wait()


def _tc_body(x_ref, o_ref):
  x = x_ref[...]
  m = jnp.max(x, axis=1, keepdims=True)
  ii = lax.broadcasted_iota(jnp.int32, x.shape, 1)
  idx = jnp.min(jnp.where(x == m, ii, _BIG), axis=1)
  at_top = idx < SIZE
  at_left = (idx > 0) & ((idx & (SIZE - 1)) == 0)
  p0 = jnp.where(at_top, 0.0, jnp.where(at_left, 1.0, 0.5))
  p1 = jnp.where(at_top, 1.0, jnp.where(at_left, 0.0, 0.5))
  z = jnp.zeros_like(p0)
  o_ref[...] = jnp.stack([p0, p1, z], axis=1)


@jax.jit
def kernel(s):
  sc_flat = pl.kernel(
      _row_kernel,
      out_type=jax.ShapeDtypeStruct((BSC * 3,), jnp.float32),
      mesh=plsc.VectorSubcoreMesh(core_axis_name="c", subcore_axis_name="s"),
      scratch_types=[
          pltpu.VMEM((CH, D), jnp.float32),
          pltpu.VMEM((CH, D), jnp.float32),
          pltpu.VMEM((RPW * 3,), jnp.float32),
          pltpu.SemaphoreType.DMA,
          pltpu.SemaphoreType.DMA,
          pltpu.SemaphoreType.DMA,
      ],
  )(s)
  tc_probs = pl.pallas_call(
      _tc_body,
      grid=((BATCH - BSC) // TCB,),
      in_specs=[pl.BlockSpec((TCB, D), lambda i: (i + BSC // TCB, 0))],
      out_specs=pl.BlockSpec((TCB, 3), lambda i: (i, 0)),
      out_shape=jax.ShapeDtypeStruct((BATCH - BSC, 3), jnp.float32),
  )(s)
  return jnp.concatenate([sc_flat.reshape(BSC, 3), tc_probs], axis=0)


# speedup vs baseline: 1.5844x; 1.0008x over previous
import functools

import jax
import jax.numpy as jnp
from jax import lax
from jax.experimental import pallas as pl
from jax.experimental.pallas import tpu as pltpu
from jax.experimental.pallas import tpu_sc as plsc

BATCH = 4096
D = 4096
SIZE = 64
NC = 2
NS = 16
NW = NC * NS
BSC = 1536
RPW = BSC // NW
CH = 8
NCHUNK = RPW // CH
NPAIR = NCHUNK // 2
L = 16
UNROLL = 16
NGROUP = D // (UNROLL * L)
TCB = 256
_BIG = 1 << 30


def _row_kernel(s_hbm, out_hbm, buf, o3, sem0, sem1, osem):
  wid = lax.axis_index("s") * NC + lax.axis_index("c")
  base = wid * RPW
  iota = lax.iota(jnp.int32, L)

  def chunk_copy(g, half, sem):
    return pltpu.make_async_copy(
        s_hbm.at[pl.ds(base + g * CH, CH)],
        buf.at[pl.ds(half * CH, CH)], sem)

  def process_row(rr, accs):
    acc0, acc1 = accs

    def step(i, carry):
      ba, ia, bb, ib = carry
      ibase = jnp.broadcast_to(i, (L,))
      for u in range(0, UNROLL, 2):
        va = buf[rr, pl.ds((i * UNROLL + u) * L, L)]
        vb = buf[rr, pl.ds((i * UNROLL + u + 1) * L, L)]
        ma = va > ba
        mb = vb > bb
        ba = jnp.where(ma, va, ba)
        ia = jnp.where(ma, ibase, ia)
        bb = jnp.where(mb, vb, bb)
        ib = jnp.where(mb, ibase, ib)
      return ba, ia, bb, ib

    ninf = jnp.full((L,), -jnp.inf, jnp.float32)
    zi = jnp.zeros((L,), jnp.int32)
    ba, ia, bb, ib = lax.fori_loop(0, NGROUP, step, (ninf, zi, ninf, zi))
    takeb = (bb > ba) | ((bb == ba) & (ib < ia))
    best = jnp.where(takeb, bb, ba)
    besti = jnp.where(takeb, ib, ia)

    m = best
    for k in (8, 4, 2, 1):
      m = jnp.maximum(m, m.at[iota ^ k].get(mode="promise_in_bounds"))
    candg = jnp.where(best == m, besti, _BIG)
    for k in (8, 4, 2, 1):
      candg = jnp.minimum(candg,
                          candg.at[iota ^ k].get(mode="promise_in_bounds"))
    gstar = candg[0]
    gcol = gstar * (UNROLL * L)
    pos_a = jnp.full((L,), _BIG, jnp.int32)
    pos_b = jnp.full((L,), _BIG, jnp.int32)
    for u in range(0, UNROLL, 2):
      va = buf[rr, pl.ds(gcol + u * L, L)]
      vb = buf[rr, pl.ds(gcol + (u + 1) * L, L)]
      pos_a = jnp.minimum(pos_a, jnp.where(va == m, iota + u * L, pos_a))
      pos_b = jnp.minimum(pos_b,
                          jnp.where(vb == m, iota + (u + 1) * L, pos_b))
    pos = jnp.minimum(pos_a, pos_b)
    for k in (8, 4, 2, 1):
      pos = jnp.minimum(pos, pos.at[iota ^ k].get(mode="promise_in_bounds"))
    idx = pos + jnp.broadcast_to(gcol, (L,))

    at_top = idx < SIZE
    at_left = (idx > 0) & ((idx & (SIZE - 1)) == 0)
    p0 = jnp.where(at_top, 0.0, jnp.where(at_left, 1.0, 0.5))
    p1 = jnp.where(at_top, 1.0, jnp.where(at_left, 0.0, 0.5))

    lanemask = iota == rr
    acc0 = jnp.where(lanemask, p0, acc0)
    acc1 = jnp.where(lanemask, p1, acc1)
    return acc0, acc1

  chunk_copy(0, 0, sem0).start()
  chunk_copy(1, 1, sem1).start()

  def chunk_iter(g, accs):
    even = (g & 1) == 0

    @pl.when(even)
    def _():
      chunk_copy(g, 0, sem0).wait()

    @pl.when(jnp.logical_not(even))
    def _():
      chunk_copy(g, 1, sem1).wait()

    rowbase = (g & 1) * CH
    acc0 = jnp.where(even, 0.0, accs[0])
    acc1 = jnp.where(even, 0.0, accs[1])
    acc0, acc1 = lax.fori_loop(
        0, CH, lambda rr, a: process_row(rowbase + rr, a), (acc0, acc1))

    @pl.when(even & (g + 2 < NCHUNK))
    def _():
      chunk_copy(g + 2, 0, sem0).start()

    @pl.when(jnp.logical_not(even) & (g + 2 < NCHUNK))
    def _():
      chunk_copy(g + 2, 1, sem1).start()

    @pl.when(jnp.logical_not(even))
    def _():
      for j in range(3):
        q = iota + (L * j)
        rowsel = (q * 21846) >> 16
        colsel = q - rowsel * 3
        g0 = acc0.at[rowsel].get(mode="promise_in_bounds")
        g1 = acc1.at[rowsel].get(mode="promise_in_bounds")
        outj = jnp.where(colsel == 0, g0, jnp.where(colsel == 1, g1, 0.0))
        o3[pl.ds((g >> 1) * (2 * CH * 3) + j * L, L)] = outj

    return acc0, acc1

  lax.fori_loop(0, NCHUNK, chunk_iter,
                (jnp.zeros((L,), jnp.float32), jnp.zeros((L,), jnp.float32)))

  pltpu.make_async_copy(
      o3, out_hbm.at[pl.ds(base * 3, RPW * 3)], osem).start()
  pltpu.make_async_copy(
      o3, out_hbm.at[pl.ds(base * 3, RPW * 3)], osem).wait()


def _tc_body(x_ref, o_ref):
  x = x_ref[...]
  m = jnp.max(x, axis=1, keepdims=True)
  ii = lax.broadcasted_iota(jnp.int32, x.shape, 1)
  idx = jnp.min(jnp.where(x == m, ii, _BIG), axis=1)
  at_top = idx < SIZE
  at_left = (idx > 0) & ((idx & (SIZE - 1)) == 0)
  p0 = jnp.where(at_top, 0.0, jnp.where(at_left, 1.0, 0.5))
  p1 = jnp.where(at_top, 1.0, jnp.where(at_left, 0.0, 0.5))
  z = jnp.zeros_like(p0)
  o_ref[...] = jnp.stack([p0, p1, z], axis=1)


@jax.jit
def kernel(s):
  sc_flat = pl.kernel(
      _row_kernel,
      out_type=jax.ShapeDtypeStruct((BSC * 3,), jnp.float32),
      mesh=plsc.VectorSubcoreMesh(core_axis_name="c", subcore_axis_name="s"),
      scratch_types=[
          pltpu.VMEM((2 * CH, D), jnp.float32),
          pltpu.VMEM((RPW * 3,), jnp.float32),
          pltpu.SemaphoreType.DMA,
          pltpu.SemaphoreType.DMA,
          pltpu.SemaphoreType.DMA,
      ],
  )(s)
  tc_probs = pl.pallas_call(
      _tc_body,
      grid=((BATCH - BSC) // TCB,),
      in_specs=[pl.BlockSpec((TCB, D), lambda i: (i + BSC // TCB, 0))],
      out_specs=pl.BlockSpec((TCB, 3), lambda i: (i, 0)),
      out_shape=jax.ShapeDtypeStruct((BATCH - BSC, 3), jnp.float32),
  )(s)
  return jnp.concatenate([sc_flat.reshape(BSC, 3), tc_probs], axis=0)
